# Initial kernel scaffold; baseline (speedup 1.0000x reference)
import jax
import jax.numpy as jnp
from jax.experimental import pallas as pl


def kernel(features, rois):
    raise NotImplementedError("write your pallas kernel here")


# speedup vs baseline: 16.7822x; 16.7822x over previous
import functools

import jax
import jax.numpy as jnp
from jax import lax
from jax.experimental import pallas as pl
from jax.experimental.pallas import tpu as pltpu
from jax.experimental.pallas import tpu_sc as plsc

_RATIO = 1.0 / 32
_PH = 7
_PW = 7
_NSAMP = _PH * _PW * 4
_NROWS = 832
_NCHUNK = 13
_L = 16


def _tec_body(H, W, C, rpw, ft, roist, out,
              b_v, x1_v, y1_v, x2_v, y2_v, idx_v, w_v, rows_v, outb_v, sem):
    nc = lax.axis_size("c")
    wid = lax.axis_index("s") * nc + lax.axis_index("c")
    Wf = float(W)
    Hf = float(H)
    cchunks = C // _L

    for g in range(rpw // _L):
        gbase = wid * rpw + g * _L
        pltpu.sync_copy(roist.at[0, pl.ds(gbase, _L)], b_v)
        pltpu.sync_copy(roist.at[1, pl.ds(gbase, _L)], x1_v)
        pltpu.sync_copy(roist.at[2, pl.ds(gbase, _L)], y1_v)
        pltpu.sync_copy(roist.at[3, pl.ds(gbase, _L)], x2_v)
        pltpu.sync_copy(roist.at[4, pl.ds(gbase, _L)], y2_v)

        def roi_body(r, _):
            ridx = jnp.full((_L,), r, jnp.int32)
            bi = plsc.load_gather(b_v, [ridx]).astype(jnp.int32)
            bx1 = jnp.clip(plsc.load_gather(x1_v, [ridx]) * _RATIO, 0.0, Wf)
            by1 = jnp.clip(plsc.load_gather(y1_v, [ridx]) * _RATIO, 0.0, Hf)
            bx2 = jnp.clip(plsc.load_gather(x2_v, [ridx]) * _RATIO, 0.0, Wf)
            by2 = jnp.clip(plsc.load_gather(y2_v, [ridx]) * _RATIO, 0.0, Hf)
            rvf = jnp.where((bx2 > bx1) & (by2 > by1), 1.0, 0.0)
            bw = (bx2 - bx1) * (1.0 / _PW)
            bh = (by2 - by1) * (1.0 / _PH)
            base = bi * (H * W)

            for ch in range(_NCHUNK):
                t = lax.iota(jnp.int32, _L) + ch * _L
                ph = t // (_PW * 4)
                rem = t % (_PW * 4)
                pw = rem // 4
                my = (rem % 4) // 2
                mx = rem % 2
                phf = ph.astype(jnp.float32)
                pwf = pw.astype(jnp.float32)
                cy1 = jnp.clip(by1 + phf * bh, 0.0, Hf)
                cy2 = jnp.clip(by1 + (phf + 1.0) * bh, 0.0, Hf)
                cx1 = jnp.clip(bx1 + pwf * bw, 0.0, Wf)
                cx2 = jnp.clip(bx1 + (pwf + 1.0) * bw, 0.0, Wf)
                bvf = jnp.where((cy2 > cy1) & (cx2 > cx1), rvf, 0.0)
                sy = cy1 + bh * 0.25 + my.astype(jnp.float32) * (bh * 0.5)
                sx = cx1 + bw * 0.25 + mx.astype(jnp.float32) * (bw * 0.5)
                y1i = jnp.clip(sy.astype(jnp.int32), 0, H - 1)
                y2i = jnp.minimum(y1i + 1, H - 1)
                x1i = jnp.clip(sx.astype(jnp.int32), 0, W - 1)
                x2i = jnp.minimum(x1i + 1, W - 1)
                wy1 = sy - y1i.astype(jnp.float32)
                wy2 = y2i.astype(jnp.float32) - sy
                wx1 = sx - x1i.astype(jnp.float32)
                wx2 = x2i.astype(jnp.float32) - sx
                corners = (
                    (y1i, x1i, wy2 * wx2),
                    (y1i, x2i, wy2 * wx1),
                    (y2i, x1i, wy1 * wx2),
                    (y2i, x2i, wy1 * wx1),
                )
                rowc = jnp.full((_L,), ch, jnp.int32)
                pcol0 = (t - ch * _L) * 4
                for k, (yy, xx, ww) in enumerate(corners):
                    rowidx = base + yy * W + xx
                    plsc.store_scatter(idx_v, [rowc, pcol0 + k], rowidx)
                    plsc.store_scatter(w_v, [(t * 4) + k], ww * bvf)

            cps = [
                pltpu.async_copy(ft.at[idx_v.at[j]],
                                 rows_v.at[pl.ds(j * 64, 64)], sem)
                for j in range(_NCHUNK)
            ]
            for cp in cps:
                cp.wait()

            def bin_body(b, _):
                r0 = b * 16
                ws = [
                    plsc.load_gather(w_v, [jnp.full((_L,), r0 + i, jnp.int32)])
                    for i in range(16)
                ]
                for cc in range(cchunks):
                    acc = None
                    for k in range(4):
                        s = None
                        for c in range(4):
                            row = rows_v[r0 + k * 4 + c, pl.ds(cc * _L, _L)]
                            term = ws[k * 4 + c] * row
                            s = term if s is None else s + term
                        acc = s if acc is None else jnp.maximum(acc, s)
                    outb_v[b, pl.ds(cc * _L, _L)] = acc
                return 0

            lax.fori_loop(0, _PH * _PW, bin_body, 0)
            pltpu.sync_copy(outb_v, out.at[gbase + r])
            return 0

        lax.fori_loop(0, _L, roi_body, 0)


def kernel(features, rois):
    B, C, H, W = features.shape
    N = rois.shape[0]
    nbins = _PH * _PW
    ft = jnp.transpose(features, (0, 2, 3, 1)).reshape(B * H * W, C)
    npad = -(-N // 512) * 512
    rpw = npad // 32
    rois_p = jnp.zeros((npad, 5), jnp.float32).at[:N].set(rois)
    roist = rois_p.T

    mesh = plsc.VectorSubcoreMesh(
        core_axis_name="c", subcore_axis_name="s", num_cores=2, num_subcores=16
    )
    body = functools.partial(_tec_body, H, W, C, rpw)
    call = pl.kernel(
        body,
        out_type=jax.ShapeDtypeStruct((npad, nbins, C), jnp.float32),
        mesh=mesh,
        scratch_types=[
            pltpu.VMEM((_L,), jnp.float32),
            pltpu.VMEM((_L,), jnp.float32),
            pltpu.VMEM((_L,), jnp.float32),
            pltpu.VMEM((_L,), jnp.float32),
            pltpu.VMEM((_L,), jnp.float32),
            pltpu.VMEM((_NCHUNK, 64), jnp.int32),
            pltpu.VMEM((_NROWS,), jnp.float32),
            pltpu.VMEM((_NROWS, C), jnp.float32),
            pltpu.VMEM((nbins, C), jnp.float32),
            pltpu.SemaphoreType.DMA,
        ],
        compiler_params=pltpu.CompilerParams(needs_layout_passes=False),
    )
    out = call(ft, roist)
    out = out[:N].reshape(N, _PH, _PW, C)
    return jnp.transpose(out, (0, 3, 1, 2))


# speedup vs baseline: 18.8945x; 1.1259x over previous
import functools

import jax
import jax.numpy as jnp
from jax import lax
from jax.experimental import pallas as pl
from jax.experimental.pallas import tpu as pltpu
from jax.experimental.pallas import tpu_sc as plsc

_RATIO = 1.0 / 32
_PH = 7
_PW = 7
_NROWS = 832
_NCHUNK = 13
_L = 16
_CHA = 7
_NBINA = _CHA * 4


def _tec_body(H, W, C, rpw, ft, roist, out,
              b_v, x1_v, y1_v, x2_v, y2_v, idx_v, w_v, rows_v, outb_v, sem):
    nc = lax.axis_size("c")
    wid = lax.axis_index("s") * nc + lax.axis_index("c")
    Wf = float(W)
    Hf = float(H)
    cchunks = C // _L

    for g in range(rpw // _L):
        gbase = wid * rpw + g * _L
        pltpu.sync_copy(roist.at[0, pl.ds(gbase, _L)], b_v)
        pltpu.sync_copy(roist.at[1, pl.ds(gbase, _L)], x1_v)
        pltpu.sync_copy(roist.at[2, pl.ds(gbase, _L)], y1_v)
        pltpu.sync_copy(roist.at[3, pl.ds(gbase, _L)], x2_v)
        pltpu.sync_copy(roist.at[4, pl.ds(gbase, _L)], y2_v)

        def roi_body(r, _):
            ridx = jnp.full((_L,), r, jnp.int32)
            bi = plsc.load_gather(b_v, [ridx]).astype(jnp.int32)
            bx1 = jnp.clip(plsc.load_gather(x1_v, [ridx]) * _RATIO, 0.0, Wf)
            by1 = jnp.clip(plsc.load_gather(y1_v, [ridx]) * _RATIO, 0.0, Hf)
            bx2 = jnp.clip(plsc.load_gather(x2_v, [ridx]) * _RATIO, 0.0, Wf)
            by2 = jnp.clip(plsc.load_gather(y2_v, [ridx]) * _RATIO, 0.0, Hf)
            rvf = jnp.where((bx2 > bx1) & (by2 > by1), 1.0, 0.0)
            bw = (bx2 - bx1) * (1.0 / _PW)
            bh = (by2 - by1) * (1.0 / _PH)
            base = bi * (H * W)

            for ch in range(_NCHUNK):
                t = lax.iota(jnp.int32, _L) + ch * _L
                ph = (t.astype(jnp.float32) * (1.0 / 28.0)).astype(jnp.int32)
                rem = t - ph * 28
                pw = rem >> 2
                my = (rem & 3) >> 1
                mx = rem & 1
                phf = ph.astype(jnp.float32)
                pwf = pw.astype(jnp.float32)
                myf = my.astype(jnp.float32)
                mxf = mx.astype(jnp.float32)
                cy1 = jnp.clip(by1 + phf * bh, 0.0, Hf)
                cy2 = jnp.clip(by1 + (phf + 1.0) * bh, 0.0, Hf)
                cx1 = jnp.clip(bx1 + pwf * bw, 0.0, Wf)
                cx2 = jnp.clip(bx1 + (pwf + 1.0) * bw, 0.0, Wf)
                bvf = jnp.where((cy2 > cy1) & (cx2 > cx1), rvf, 0.0)
                sy = cy1 + bh * 0.25 + myf * (bh * 0.5)
                sx = cx1 + bw * 0.25 + mxf * (bw * 0.5)
                y1i = jnp.clip(sy.astype(jnp.int32), 0, H - 1)
                y2i = jnp.minimum(y1i + 1, H - 1)
                x1i = jnp.clip(sx.astype(jnp.int32), 0, W - 1)
                x2i = jnp.minimum(x1i + 1, W - 1)
                wy1 = sy - y1i.astype(jnp.float32)
                wy2 = y2i.astype(jnp.float32) - sy
                wx1 = sx - x1i.astype(jnp.float32)
                wx2 = x2i.astype(jnp.float32) - sx
                corners = (
                    (y1i, x1i, wy2 * wx2),
                    (y1i, x2i, wy2 * wx1),
                    (y2i, x1i, wy1 * wx2),
                    (y2i, x2i, wy1 * wx1),
                )
                rowc = jnp.full((_L,), ch, jnp.int32)
                pcol0 = (t - ch * _L) * 4
                for k, (yy, xx, ww) in enumerate(corners):
                    rowidx = base + yy * W + xx
                    plsc.store_scatter(idx_v, [rowc, pcol0 + k], rowidx)
                    plsc.store_scatter(w_v, [(t * 4) + k], ww * bvf)

            cps = [
                pltpu.async_copy(ft.at[idx_v.at[j]],
                                 rows_v.at[pl.ds(j * 64, 64)], sem)
                for j in range(_NCHUNK)
            ]

            def bin_body(b, _):
                r0 = b * 16
                ws = [
                    plsc.load_gather(w_v, [jnp.full((_L,), r0 + i, jnp.int32)])
                    for i in range(16)
                ]
                for cc in range(cchunks):
                    acc = None
                    for k in range(4):
                        sv = None
                        for c in range(4):
                            row = rows_v[r0 + k * 4 + c, pl.ds(cc * _L, _L)]
                            term = ws[k * 4 + c] * row
                            sv = term if sv is None else sv + term
                        acc = sv if acc is None else jnp.maximum(acc, sv)
                    outb_v[b, pl.ds(cc * _L, _L)] = acc
                return 0

            for cp in cps[:_CHA]:
                cp.wait()
            lax.fori_loop(0, _NBINA, bin_body, 0)
            for cp in cps[_CHA:]:
                cp.wait()
            lax.fori_loop(_NBINA, _PH * _PW, bin_body, 0)
            pltpu.sync_copy(outb_v, out.at[gbase + r])
            return 0

        lax.fori_loop(0, _L, roi_body, 0)


def kernel(features, rois):
    B, C, H, W = features.shape
    N = rois.shape[0]
    nbins = _PH * _PW
    ft = jnp.transpose(features, (0, 2, 3, 1)).reshape(B * H * W, C)
    npad = -(-N // 512) * 512
    rpw = npad // 32
    rois_p = jnp.zeros((npad, 5), jnp.float32).at[:N].set(rois)
    roist = rois_p.T

    mesh = plsc.VectorSubcoreMesh(
        core_axis_name="c", subcore_axis_name="s", num_cores=2, num_subcores=16
    )
    body = functools.partial(_tec_body, H, W, C, rpw)
    call = pl.kernel(
        body,
        out_type=jax.ShapeDtypeStruct((npad, nbins, C), jnp.float32),
        mesh=mesh,
        scratch_types=[
            pltpu.VMEM((_L,), jnp.float32),
            pltpu.VMEM((_L,), jnp.float32),
            pltpu.VMEM((_L,), jnp.float32),
            pltpu.VMEM((_L,), jnp.float32),
            pltpu.VMEM((_L,), jnp.float32),
            pltpu.VMEM((_NCHUNK, 64), jnp.int32),
            pltpu.VMEM((_NROWS,), jnp.float32),
            pltpu.VMEM((_NROWS, C), jnp.float32),
            pltpu.VMEM((nbins, C), jnp.float32),
            pltpu.SemaphoreType.DMA,
        ],
        compiler_params=pltpu.CompilerParams(needs_layout_passes=False),
    )
    out = call(ft, roist)
    out = out[:N].reshape(N, _PH, _PW, C)
    return jnp.transpose(out, (0, 3, 1, 2))


# speedup vs baseline: 20.7392x; 1.0976x over previous
import functools

import jax
import jax.numpy as jnp
from jax import lax
from jax.experimental import pallas as pl
from jax.experimental.pallas import tpu as pltpu
from jax.experimental.pallas import tpu_sc as plsc

_RATIO = 1.0 / 32
_PH = 7
_PW = 7
_NROWS = 832
_NCHUNK = 13
_L = 16
_CHA = 7
_NBINA = _CHA * 4


def _tec_body(H, W, C, rpw, ft, roist, out,
              b_v, x1_v, y1_v, x2_v, y2_v, idx_v, w_v, rows_v, outb_v, sem):
    nc = lax.axis_size("c")
    wid = lax.axis_index("s") * nc + lax.axis_index("c")
    Wf = float(W)
    Hf = float(H)
    cchunks = C // _L

    for g in range(rpw // _L):
        gbase = wid * rpw + g * _L
        pltpu.sync_copy(roist.at[0, pl.ds(gbase, _L)], b_v)
        pltpu.sync_copy(roist.at[1, pl.ds(gbase, _L)], x1_v)
        pltpu.sync_copy(roist.at[2, pl.ds(gbase, _L)], y1_v)
        pltpu.sync_copy(roist.at[3, pl.ds(gbase, _L)], x2_v)
        pltpu.sync_copy(roist.at[4, pl.ds(gbase, _L)], y2_v)

        def roi_body(r, _):
            ridx = jnp.full((_L,), r, jnp.int32)
            bi = plsc.load_gather(b_v, [ridx]).astype(jnp.int32)
            bx1 = jnp.clip(plsc.load_gather(x1_v, [ridx]) * _RATIO, 0.0, Wf)
            by1 = jnp.clip(plsc.load_gather(y1_v, [ridx]) * _RATIO, 0.0, Hf)
            bx2 = jnp.clip(plsc.load_gather(x2_v, [ridx]) * _RATIO, 0.0, Wf)
            by2 = jnp.clip(plsc.load_gather(y2_v, [ridx]) * _RATIO, 0.0, Hf)
            rvf = jnp.where((bx2 > bx1) & (by2 > by1), 1.0, 0.0)
            bw = (bx2 - bx1) * (1.0 / _PW)
            bh = (by2 - by1) * (1.0 / _PH)
            base = bi * (H * W)

            for ch in range(_NCHUNK):
                t = lax.iota(jnp.int32, _L) + ch * _L
                ph = (t.astype(jnp.float32) * (1.0 / 28.0)).astype(jnp.int32)
                rem = t - ph * 28
                pw = rem >> 2
                my = (rem & 3) >> 1
                mx = rem & 1
                phf = ph.astype(jnp.float32)
                pwf = pw.astype(jnp.float32)
                myf = my.astype(jnp.float32)
                mxf = mx.astype(jnp.float32)
                cy1 = jnp.clip(by1 + phf * bh, 0.0, Hf)
                cy2 = jnp.clip(by1 + (phf + 1.0) * bh, 0.0, Hf)
                cx1 = jnp.clip(bx1 + pwf * bw, 0.0, Wf)
                cx2 = jnp.clip(bx1 + (pwf + 1.0) * bw, 0.0, Wf)
                bvf = jnp.where((cy2 > cy1) & (cx2 > cx1), rvf, 0.0)
                sy = cy1 + bh * 0.25 + myf * (bh * 0.5)
                sx = cx1 + bw * 0.25 + mxf * (bw * 0.5)
                y1i = jnp.clip(sy.astype(jnp.int32), 0, H - 1)
                y2i = jnp.minimum(y1i + 1, H - 1)
                x1i = jnp.clip(sx.astype(jnp.int32), 0, W - 1)
                x2i = jnp.minimum(x1i + 1, W - 1)
                wy1 = sy - y1i.astype(jnp.float32)
                wy2 = y2i.astype(jnp.float32) - sy
                wx1 = sx - x1i.astype(jnp.float32)
                wx2 = x2i.astype(jnp.float32) - sx
                corners = (
                    (y1i, x1i, wy2 * wx2),
                    (y1i, x2i, wy2 * wx1),
                    (y2i, x1i, wy1 * wx2),
                    (y2i, x2i, wy1 * wx1),
                )
                rowc = jnp.full((_L,), ch, jnp.int32)
                pcol0 = (t - ch * _L) * 4
                for k, (yy, xx, ww) in enumerate(corners):
                    rowidx = base + yy * W + xx
                    plsc.store_scatter(idx_v, [rowc, pcol0 + k], rowidx)
                    plsc.store_scatter(w_v, [(t * 4) + k], ww * bvf)
                pltpu.async_copy(ft.at[idx_v.at[ch]],
                                 rows_v.at[pl.ds(ch * 64, 64)], sem)

            def bin_body(b, _):
                r0 = b * 16
                ws = [
                    plsc.load_gather(w_v, [jnp.full((_L,), r0 + i, jnp.int32)])
                    for i in range(16)
                ]
                for cc in range(cchunks):
                    acc = None
                    for k in range(4):
                        sv = None
                        for c in range(4):
                            row = rows_v[r0 + k * 4 + c, pl.ds(cc * _L, _L)]
                            term = ws[k * 4 + c] * row
                            sv = term if sv is None else sv + term
                        acc = sv if acc is None else jnp.maximum(acc, sv)
                    outb_v[b, pl.ds(cc * _L, _L)] = acc
                return 0

            def chunk_body(j, _):
                pltpu.make_async_copy(ft.at[idx_v.at[j]],
                                      rows_v.at[pl.ds(j * 64, 64)],
                                      sem).wait()
                lax.fori_loop(j * 4, jnp.minimum(j * 4 + 4, _PH * _PW),
                              bin_body, 0)
                return 0

            lax.fori_loop(0, _NCHUNK, chunk_body, 0)
            pltpu.sync_copy(outb_v, out.at[gbase + r])
            return 0

        lax.fori_loop(0, _L, roi_body, 0)


def kernel(features, rois):
    B, C, H, W = features.shape
    N = rois.shape[0]
    nbins = _PH * _PW
    ft = jnp.transpose(features, (0, 2, 3, 1)).reshape(B * H * W, C)
    npad = -(-N // 512) * 512
    rpw = npad // 32
    rois_p = jnp.zeros((npad, 5), jnp.float32).at[:N].set(rois)
    roist = rois_p.T

    mesh = plsc.VectorSubcoreMesh(
        core_axis_name="c", subcore_axis_name="s", num_cores=2, num_subcores=16
    )
    body = functools.partial(_tec_body, H, W, C, rpw)
    call = pl.kernel(
        body,
        out_type=jax.ShapeDtypeStruct((npad, nbins, C), jnp.float32),
        mesh=mesh,
        scratch_types=[
            pltpu.VMEM((_L,), jnp.float32),
            pltpu.VMEM((_L,), jnp.float32),
            pltpu.VMEM((_L,), jnp.float32),
            pltpu.VMEM((_L,), jnp.float32),
            pltpu.VMEM((_L,), jnp.float32),
            pltpu.VMEM((_NCHUNK, 64), jnp.int32),
            pltpu.VMEM((_NROWS,), jnp.float32),
            pltpu.VMEM((_NROWS, C), jnp.float32),
            pltpu.VMEM((nbins, C), jnp.float32),
            pltpu.SemaphoreType.DMA,
        ],
        compiler_params=pltpu.CompilerParams(needs_layout_passes=False),
    )
    out = call(ft, roist)
    out = out[:N].reshape(N, _PH, _PW, C)
    return jnp.transpose(out, (0, 3, 1, 2))
